# baseline (speedup 1.0000x reference)
import jax
import jax.numpy as jnp
from jax import lax
from jax.experimental import pallas as pl
from jax.experimental.pallas import tpu as pltpu


def kernel(logits, noise, x, mask, unmask_idx):
    B, F, V = logits.shape
    L = unmask_idx.shape[1]

    logits_t = jnp.transpose(logits, (1, 2, 0))
    noise_t = jnp.transpose(noise, (1, 2, 0))
    x_t = x.T
    mask_t = mask.T
    umi_t = unmask_idx.T

    def tc_body(lg_ref, ns_ref, xt_ref, mt_ref, umi_ref,
                newx_ref, newm_ref, fm_ref):
        f = pl.program_id(0)
        val = lg_ref[0] + ns_ref[0]
        maxv = jnp.max(val, axis=0)
        iota_v = lax.broadcasted_iota(jnp.int32, (V, B), 0)
        amax = jnp.min(jnp.where(val == maxv[None, :], iota_v, V), axis=0)
        m = mt_ref[pl.ds(f, 1), :]
        hit = (umi_ref[pl.ds(0, 1), :] == f)
        for l in range(1, L):
            hit = hit | (umi_ref[pl.ds(l, 1), :] == f)
        diff = hit & jnp.logical_not(m)
        newx_ref[pl.ds(f, 1), :] = jnp.where(
            diff, amax[None, :], xt_ref[pl.ds(f, 1), :])
        newm_ref[pl.ds(f, 1), :] = hit | m
        fm_ref[pl.ds(f, 1), :] = jnp.where(m, 0.0, -jnp.inf)

    new_x_t, new_mask_t, fm_t = pl.pallas_call(
        tc_body,
        grid=(F,),
        in_specs=[
            pl.BlockSpec((1, V, B), lambda f: (f, 0, 0)),
            pl.BlockSpec((1, V, B), lambda f: (f, 0, 0)),
            pl.BlockSpec((F, B), lambda f: (0, 0)),
            pl.BlockSpec((F, B), lambda f: (0, 0)),
            pl.BlockSpec((L, B), lambda f: (0, 0)),
        ],
        out_specs=[
            pl.BlockSpec((F, B), lambda f: (0, 0)),
            pl.BlockSpec((F, B), lambda f: (0, 0)),
            pl.BlockSpec((F, B), lambda f: (0, 0)),
        ],
        out_shape=[
            jax.ShapeDtypeStruct((F, B), jnp.int32),
            jax.ShapeDtypeStruct((F, B), jnp.bool_),
            jax.ShapeDtypeStruct((F, B), jnp.float32),
        ],
        compiler_params=pltpu.CompilerParams(
            dimension_semantics=("arbitrary",)),
    )(logits_t, noise_t, x_t, mask_t, umi_t)
    return new_x_t.T, new_mask_t.T, fm_t.T
